# baseline (speedup 1.0000x reference)
import functools

import jax
import jax.numpy as jnp
from jax import lax
from jax.experimental import pallas as pl
from jax.experimental.pallas import tpu as pltpu
from jax.experimental.pallas import tpu_sc as plsc

_NG = 100000
_NM = 10000
_NE = 500000
_D = 64
_W = 128

_NC = 2
_NS = 16
_NW = _NC * _NS
_CH = 128
_PER_W = 15872
_CHUNKS = _PER_W // _CH
_NEP = _PER_W * _NW
_ACC = 10240
_APT = _ACC // _NS

_BLE = 2000
_BLN = 2000


def _ln(h):
    mu = jnp.mean(h, axis=-1, keepdims=True)
    var = jnp.mean((h - mu) ** 2, axis=-1, keepdims=True)
    return (h - mu) * lax.rsqrt(var + 1e-5)


def _dot(a, b):
    return jnp.dot(a, b, preferred_element_type=jnp.float32)


def _wspec(shape):
    return pl.BlockSpec(shape, lambda i: (0,) * len(shape))


def _mesh_body(x_ref, w0, b0, w1, b1, g, b, w0i, c0, me_ref, ta_ref):
    x = x_ref[...]
    h = jnp.maximum(_dot(x, w0[...]) + b0[...], 0.0)
    h = _dot(h, w1[...]) + b1[...]
    me = _ln(h) * g[...] + b[...]
    me_ref[...] = me
    a = _dot(me, w0i[...]) + c0[...]
    ta_ref[...] = jnp.concatenate([a, jnp.zeros_like(a)], axis=1)


def _gridhead_body(x_ref, w0, b0, w1, b1, g, b, w0j, tb_ref):
    x = x_ref[...]
    h = jnp.maximum(_dot(x, w0[...]) + b0[...], 0.0)
    h = _dot(h, w1[...]) + b1[...]
    ge = _ln(h) * g[...] + b[...]
    tb = _dot(ge, w0j[...])
    tb_ref[...] = jnp.concatenate([jnp.zeros_like(tb), tb], axis=1)


def _edge_msg_body(p_ref, g_in_ref, u1, u1b, gg, gb, out_ref):
    gv = g_in_ref[...]
    h1 = p_ref[...] + gv[:, :_D] + gv[:, _D:]
    y = jnp.maximum(h1, 0.0)
    y = _dot(y, u1[...]) + u1b[...]
    msg = _ln(y) * gg[...] + gb[...]
    out_ref[...] = jnp.concatenate([msg, jnp.zeros_like(msg)], axis=1)


def _grid_body(x_ref, w0, b0, w1, b1, g, b, v0, d0, v1, d1, gf, bf, out_ref):
    x = x_ref[...]
    h = jnp.maximum(_dot(x, w0[...]) + b0[...], 0.0)
    h = _dot(h, w1[...]) + b1[...]
    ge = _ln(h) * g[...] + b[...]
    f = jnp.maximum(_dot(ge, v0[...]) + d0[...], 0.0)
    f = _dot(f, v1[...]) + d1[...]
    out_ref[...] = ge + _ln(f) * gf[...] + bf[...]


def _edge_pre_body(e_ref, we0, be0, we1, be1, w0e, p_ref):
    e = e_ref[...]
    h = jnp.maximum(_dot(e, we0[...]) + be0[...], 0.0)
    h = _dot(h, we1[...]) + be1[...]
    nh = _ln(h)
    p_ref[...] = _dot(nh, w0e[...])


def _mesh_out_body(me_ref, p0_ref, p1_ref, w0m, w0a, b0, w1, b1, g, b,
                   out_ref):
    me = me_ref[...]
    aggr = p0_ref[...] + p1_ref[...]
    h = jnp.maximum(_dot(me, w0m[...]) + _dot(aggr, w0a[...]) + b0[...], 0.0)
    h = _dot(h, w1[...]) + b1[...]
    out_ref[...] = me + _ln(h) * g[...] + b[...]


def _run_rows(body, n_rows, blk, n_blocked, out_widths, args):
    in_specs = []
    for i, a in enumerate(args):
        if i < n_blocked:
            in_specs.append(pl.BlockSpec((blk, a.shape[1]), lambda i: (i, 0)))
        else:
            in_specs.append(_wspec(a.shape))
    out = pl.pallas_call(
        body,
        grid=(n_rows // blk,),
        in_specs=in_specs,
        out_shape=[jax.ShapeDtypeStruct((n_rows, w), jnp.float32)
                   for w in out_widths],
        out_specs=[pl.BlockSpec((blk, w), lambda i: (i, 0))
                   for w in out_widths],
    )(*args)
    return out


_SC_MESH = plsc.VectorSubcoreMesh(core_axis_name="c", subcore_axis_name="s")


@functools.partial(
    pl.kernel,
    out_type=jax.ShapeDtypeStruct((_NEP, _W), jnp.float32),
    mesh=_SC_MESH,
    scratch_types=[
        pltpu.VMEM((_CHUNKS, _CH), jnp.int32),
        pltpu.VMEM((_CHUNKS, _CH), jnp.int32),
        pltpu.VMEM((_CH, _W), jnp.float32),
        pltpu.VMEM((_CH, _W), jnp.float32),
        pltpu.VMEM((_CH, _W), jnp.float32),
        pltpu.VMEM((_CH, _W), jnp.float32),
        pltpu.SemaphoreType.DMA,
        pltpu.SemaphoreType.DMA,
    ],
)
def _sc_gather(ta_hbm, tb_hbm, recv_hbm, send_hbm, g_hbm,
               ridx_all, sidx_all, ra0, ra1, rb0, rb1, sem_a, sem_b):
    c = lax.axis_index("c")
    s = lax.axis_index("s")
    w = s * _NC + c
    base = w * _PER_W
    ta = ta_hbm.at[c]
    tb = tb_hbm.at[c]
    pltpu.sync_copy(recv_hbm.at[w], ridx_all)
    pltpu.sync_copy(send_hbm.at[w], sidx_all)
    bufs_a = (ra0, ra1)
    bufs_b = (rb0, rb1)
    pltpu.async_copy(ta.at[ridx_all.at[0]], ra0, sem_a)
    pltpu.async_copy(tb.at[sidx_all.at[0]], rb0, sem_b)

    def body(i, carry):
        for b in range(2):
            j = i * 2 + b
            ra, rb = bufs_a[b], bufs_b[b]
            na, nb = bufs_a[1 - b], bufs_b[1 - b]
            jn = j + 1

            @pl.when(jn < _CHUNKS)
            def _():
                pltpu.async_copy(ta.at[ridx_all.at[jn]], na, sem_a)
                pltpu.async_copy(tb.at[sidx_all.at[jn]], nb, sem_b)

            pltpu.make_async_copy(ta.at[ridx_all.at[j]], ra, sem_a).wait()
            pltpu.make_async_copy(tb.at[sidx_all.at[j]], rb, sem_b).wait()

            def merge(r, cc):
                for k in range(_D // 16):
                    sl = pl.ds(_D + 16 * k, 16)
                    ra[r, sl] = rb[r, sl]
                return cc

            lax.fori_loop(0, _CH, merge, 0)
            pltpu.sync_copy(ra, g_hbm.at[pl.ds(base + j * _CH, _CH)])
        return carry

    lax.fori_loop(0, _CHUNKS // 2, body, 0)


@functools.partial(
    pl.kernel,
    out_type=jax.ShapeDtypeStruct((_NC, _ACC, _W), jnp.float32),
    mesh=_SC_MESH,
    scratch_types=[
        pltpu.VMEM((_CHUNKS, _CH), jnp.int32),
        pltpu.VMEM((_CH, _W), jnp.float32),
        pltpu.VMEM((_CH, _W), jnp.float32),
        pltpu.VMEM_SHARED((_ACC, _W), jnp.float32),
        pltpu.SemaphoreType.DMA,
    ],
)
def _sc_scatter(msg_hbm, recv_hbm, zeros_hbm, out_hbm,
                ridx_all, r0, r1, acc, sem_m):
    c = lax.axis_index("c")
    s = lax.axis_index("s")
    pltpu.sync_copy(zeros_hbm, acc.at[pl.ds(s * _APT, _APT)])
    w = s * _NC + c
    base = w * _PER_W
    pltpu.sync_copy(recv_hbm.at[w], ridx_all)
    plsc.subcore_barrier()
    bufs = (r0, r1)
    pltpu.async_copy(msg_hbm.at[pl.ds(base, _CH)], r0, sem_m)

    def body(i, carry):
        for b in range(2):
            j = i * 2 + b
            rcur = bufs[b]
            rnext = bufs[1 - b]
            jn = j + 1

            @pl.when(jn < _CHUNKS)
            def _():
                pltpu.async_copy(msg_hbm.at[pl.ds(base + jn * _CH, _CH)],
                                 rnext, sem_m)

            pltpu.make_async_copy(msg_hbm.at[pl.ds(base, _CH)], rcur,
                                  sem_m).wait()
            pltpu.sync_copy(rcur, acc.at[ridx_all.at[j]], add=True)
        return carry

    lax.fori_loop(0, _CHUNKS // 2, body, 0)
    plsc.subcore_barrier()
    pltpu.sync_copy(acc.at[pl.ds(s * _APT, _APT)],
                    out_hbm.at[c].at[pl.ds(s * _APT, _APT)])


def kernel(input_grid_nodes, input_mesh_nodes, input_edge_attr, params,
           edge_index):
    p_g = params["grid_mlp"]
    p_m = params["mesh_mlp"]
    p_e = params["edges_mlp"]
    p_ge = params["gnn_edges"]
    p_gn = params["gnn_nodes"]
    p_gf = params["grid_mlp_final"]

    f32 = jnp.float32
    row = lambda v: v.reshape(1, _D).astype(f32)

    w0 = p_ge["Ws"][0]
    w0i, w0j, w0e = w0[:_D], w0[_D:2 * _D], w0[2 * _D:]
    w0e_eff = p_e["ln_g"][:, None] * w0e
    c0 = (p_e["ln_b"] @ w0e + p_ge["bs"][0]).reshape(1, _D)

    mesh8 = jnp.pad(input_mesh_nodes, ((0, 0), (0, 5)))
    wm0 = jnp.pad(p_m["Ws"][0], ((0, 5), (0, 0)))

    mesh_emb, table_a = _run_rows(
        _mesh_body, _NM, _BLN, 1, (_D, _W),
        (mesh8, wm0, row(p_m["bs"][0]), p_m["Ws"][1], row(p_m["bs"][1]),
         row(p_m["ln_g"]), row(p_m["ln_b"]), w0i, c0))

    (table_b,) = _run_rows(
        _gridhead_body, _NM, _BLN, 1, (_W,),
        (input_grid_nodes[:_NM], p_g["Ws"][0], row(p_g["bs"][0]),
         p_g["Ws"][1], row(p_g["bs"][1]), row(p_g["ln_g"]),
         row(p_g["ln_b"]), w0j))

    shp = (_NW, _CHUNKS, _CH)
    send = jnp.pad(edge_index[0], (0, _NEP - _NE)).reshape(shp)
    recv = jnp.pad(edge_index[1], (0, _NEP - _NE)).reshape(shp)
    recv_s = jnp.pad(edge_index[1], (0, _NEP - _NE),
                     constant_values=_NM).reshape(shp)
    table_a2 = jnp.stack([table_a, table_a])
    table_b2 = jnp.stack([table_b, table_b])
    g_edges = _sc_gather(table_a2, table_b2, recv, send)

    (edge_p,) = _run_rows(
        _edge_pre_body, _NE, _BLE, 1, (_D,),
        (input_edge_attr, p_e["Ws"][0], row(p_e["bs"][0]), p_e["Ws"][1],
         row(p_e["bs"][1]), w0e_eff))

    eargs = (edge_p, g_edges, p_ge["Ws"][1], row(p_ge["bs"][1]),
             row(p_ge["ln_g"]), row(p_ge["ln_b"]))
    in_specs = [pl.BlockSpec((_BLE, _D), lambda i: (i, 0)),
                pl.BlockSpec((_BLE, _W), lambda i: (i, 0))]
    in_specs += [_wspec(a.shape) for a in eargs[2:]]
    msg = pl.pallas_call(
        _edge_msg_body,
        grid=(_NE // _BLE,),
        in_specs=in_specs,
        out_shape=jax.ShapeDtypeStruct((_NEP, _W), jnp.float32),
        out_specs=pl.BlockSpec((_BLE, _W), lambda i: (i, 0)),
    )(*eargs)

    partials = _sc_scatter(msg, recv_s, jnp.zeros((_APT, _W), f32))

    wn0 = p_gn["Ws"][0]
    (latent_mesh,) = _run_rows(
        _mesh_out_body, _NM, _BLN, 3, (_D,),
        (mesh_emb, partials[0, :_NM, :_D], partials[1, :_NM, :_D],
         wn0[:_D], wn0[_D:], row(p_gn["bs"][0]), p_gn["Ws"][1],
         row(p_gn["bs"][1]), row(p_gn["ln_g"]), row(p_gn["ln_b"])))

    (latent_grid,) = _run_rows(
        _grid_body, _NG, _BLN, 1, (_D,),
        (input_grid_nodes, p_g["Ws"][0], row(p_g["bs"][0]), p_g["Ws"][1],
         row(p_g["bs"][1]), row(p_g["ln_g"]), row(p_g["ln_b"]),
         p_gf["Ws"][0], row(p_gf["bs"][0]), p_gf["Ws"][1], row(p_gf["bs"][1]),
         row(p_gf["ln_g"]), row(p_gf["ln_b"])))

    return (latent_grid, latent_mesh)


# speedup vs baseline: 1.0359x; 1.0359x over previous
import functools

import jax
import jax.numpy as jnp
from jax import lax
from jax.experimental import pallas as pl
from jax.experimental.pallas import tpu as pltpu
from jax.experimental.pallas import tpu_sc as plsc

_NG = 100000
_NM = 10000
_NE = 500000
_D = 64
_W = 128

_NC = 2
_NS = 16
_NW = _NC * _NS
_CH = 128
_PER_W = 15872
_CHUNKS = _PER_W // _CH
_NEP = _PER_W * _NW
_ACC = 10240
_APT = _ACC // _NS

_BLE = 2000
_BLN = 2000


def _ln(h):
    mu = jnp.mean(h, axis=-1, keepdims=True)
    var = jnp.mean((h - mu) ** 2, axis=-1, keepdims=True)
    return (h - mu) * lax.rsqrt(var + 1e-5)


def _dot(a, b):
    return jnp.dot(a, b, preferred_element_type=jnp.float32)


def _wspec(shape):
    return pl.BlockSpec(shape, lambda i: (0,) * len(shape))


def _mesh_body(x_ref, w0, b0, w1, b1, g, b, w0i, c0, me_ref, ta_ref):
    x = x_ref[...]
    h = jnp.maximum(_dot(x, w0[...]) + b0[...], 0.0)
    h = _dot(h, w1[...]) + b1[...]
    me = _ln(h) * g[...] + b[...]
    me_ref[...] = me
    a = _dot(me, w0i[...]) + c0[...]
    ta_ref[...] = jnp.concatenate([a, jnp.zeros_like(a)], axis=1)


def _gridhead_body(x_ref, w0, b0, w1, b1, g, b, w0j, tb_ref):
    x = x_ref[...]
    h = jnp.maximum(_dot(x, w0[...]) + b0[...], 0.0)
    h = _dot(h, w1[...]) + b1[...]
    ge = _ln(h) * g[...] + b[...]
    tb = _dot(ge, w0j[...])
    tb_ref[...] = jnp.concatenate([jnp.zeros_like(tb), tb], axis=1)


def _edge_msg_body(e_ref, g_in_ref, we0, be0, we1, be1, w0e, u1, u1b, gg, gb,
                   out_ref):
    e = e_ref[...]
    h = jnp.maximum(_dot(e, we0[...]) + be0[...], 0.0)
    h = _dot(h, we1[...]) + be1[...]
    nh = _ln(h)
    gv = g_in_ref[...]
    h1 = _dot(nh, w0e[...]) + gv[:, :_D] + gv[:, _D:]
    y = jnp.maximum(h1, 0.0)
    y = _dot(y, u1[...]) + u1b[...]
    out_ref[...] = _ln(y) * gg[...] + gb[...]


def _grid_body(x_ref, w0, b0, w1, b1, g, b, v0, d0, v1, d1, gf, bf, out_ref):
    x = x_ref[...]
    h = jnp.maximum(_dot(x, w0[...]) + b0[...], 0.0)
    h = _dot(h, w1[...]) + b1[...]
    ge = _ln(h) * g[...] + b[...]
    f = jnp.maximum(_dot(ge, v0[...]) + d0[...], 0.0)
    f = _dot(f, v1[...]) + d1[...]
    out_ref[...] = ge + _ln(f) * gf[...] + bf[...]


def _mesh_out_body(me_ref, p0_ref, p1_ref, w0m, w0a, b0, w1, b1, g, b,
                   out_ref):
    me = me_ref[...]
    aggr = p0_ref[...] + p1_ref[...]
    h = jnp.maximum(_dot(me, w0m[...]) + _dot(aggr, w0a[...]) + b0[...], 0.0)
    h = _dot(h, w1[...]) + b1[...]
    out_ref[...] = me + _ln(h) * g[...] + b[...]


def _run_rows(body, n_rows, blk, n_blocked, out_widths, args):
    in_specs = []
    for i, a in enumerate(args):
        if i < n_blocked:
            in_specs.append(pl.BlockSpec((blk, a.shape[1]), lambda i: (i, 0)))
        else:
            in_specs.append(_wspec(a.shape))
    out = pl.pallas_call(
        body,
        grid=(n_rows // blk,),
        in_specs=in_specs,
        out_shape=[jax.ShapeDtypeStruct((n_rows, w), jnp.float32)
                   for w in out_widths],
        out_specs=[pl.BlockSpec((blk, w), lambda i: (i, 0))
                   for w in out_widths],
    )(*args)
    return out


_SC_MESH = plsc.VectorSubcoreMesh(core_axis_name="c", subcore_axis_name="s")


@functools.partial(
    pl.kernel,
    out_type=jax.ShapeDtypeStruct((_NEP, _W), jnp.float32),
    mesh=_SC_MESH,
    scratch_types=[
        pltpu.VMEM((_CHUNKS, _CH), jnp.int32),
        pltpu.VMEM((_CHUNKS, _CH), jnp.int32),
        pltpu.VMEM((_CH, _W), jnp.float32),
        pltpu.VMEM((_CH, _W), jnp.float32),
        pltpu.VMEM((_CH, _W), jnp.float32),
        pltpu.VMEM((_CH, _W), jnp.float32),
        pltpu.SemaphoreType.DMA,
        pltpu.SemaphoreType.DMA,
    ],
)
def _sc_gather(ta_hbm, tb_hbm, recv_hbm, send_hbm, g_hbm,
               ridx_all, sidx_all, ra0, ra1, rb0, rb1, sem_a, sem_b):
    c = lax.axis_index("c")
    s = lax.axis_index("s")
    w = s * _NC + c
    base = w * _PER_W
    ta = ta_hbm
    tb = tb_hbm
    pltpu.sync_copy(recv_hbm.at[w], ridx_all)
    pltpu.sync_copy(send_hbm.at[w], sidx_all)
    bufs_a = (ra0, ra1)
    bufs_b = (rb0, rb1)
    pltpu.async_copy(ta.at[ridx_all.at[0]], ra0, sem_a)
    pltpu.async_copy(tb.at[sidx_all.at[0]], rb0, sem_b)

    def body(i, carry):
        for b in range(2):
            j = i * 2 + b
            ra, rb = bufs_a[b], bufs_b[b]
            na, nb = bufs_a[1 - b], bufs_b[1 - b]
            jn = j + 1

            @pl.when(jn < _CHUNKS)
            def _():
                pltpu.async_copy(ta.at[ridx_all.at[jn]], na, sem_a)
                pltpu.async_copy(tb.at[sidx_all.at[jn]], nb, sem_b)

            pltpu.make_async_copy(ta.at[ridx_all.at[j]], ra, sem_a).wait()
            pltpu.make_async_copy(tb.at[sidx_all.at[j]], rb, sem_b).wait()

            def merge(r, cc):
                for k in range(_D // 16):
                    sl = pl.ds(_D + 16 * k, 16)
                    ra[r, sl] = rb[r, sl]
                return cc

            lax.fori_loop(0, _CH, merge, 0)
            pltpu.sync_copy(ra, g_hbm.at[pl.ds(base + j * _CH, _CH)])
        return carry

    lax.fori_loop(0, _CHUNKS // 2, body, 0)


@functools.partial(
    pl.kernel,
    out_type=jax.ShapeDtypeStruct((_NC, _ACC, _D), jnp.float32),
    mesh=_SC_MESH,
    scratch_types=[
        pltpu.VMEM((_CHUNKS, _CH), jnp.int32),
        pltpu.VMEM((_CH, _D), jnp.float32),
        pltpu.VMEM((_CH, _D), jnp.float32),
        pltpu.VMEM_SHARED((_ACC, _D), jnp.float32),
        pltpu.SemaphoreType.DMA,
    ],
)
def _sc_scatter(msg_hbm, recv_hbm, zeros_hbm, out_hbm,
                ridx_all, r0, r1, acc, sem_m):
    c = lax.axis_index("c")
    s = lax.axis_index("s")
    pltpu.sync_copy(zeros_hbm, acc.at[pl.ds(s * _APT, _APT)])
    w = s * _NC + c
    base = w * _PER_W
    pltpu.sync_copy(recv_hbm.at[w], ridx_all)
    plsc.subcore_barrier()
    bufs = (r0, r1)
    pltpu.async_copy(msg_hbm.at[pl.ds(base, _CH)], r0, sem_m)

    def body(i, carry):
        for b in range(2):
            j = i * 2 + b
            rcur = bufs[b]
            rnext = bufs[1 - b]
            jn = j + 1

            @pl.when(jn < _CHUNKS)
            def _():
                pltpu.async_copy(msg_hbm.at[pl.ds(base + jn * _CH, _CH)],
                                 rnext, sem_m)

            pltpu.make_async_copy(msg_hbm.at[pl.ds(base, _CH)], rcur,
                                  sem_m).wait()
            pltpu.sync_copy(rcur, acc.at[ridx_all.at[j]], add=True)
        return carry

    lax.fori_loop(0, _CHUNKS // 2, body, 0)
    plsc.subcore_barrier()
    pltpu.sync_copy(acc.at[pl.ds(s * _APT, _APT)],
                    out_hbm.at[c].at[pl.ds(s * _APT, _APT)])


def kernel(input_grid_nodes, input_mesh_nodes, input_edge_attr, params,
           edge_index):
    p_g = params["grid_mlp"]
    p_m = params["mesh_mlp"]
    p_e = params["edges_mlp"]
    p_ge = params["gnn_edges"]
    p_gn = params["gnn_nodes"]
    p_gf = params["grid_mlp_final"]

    f32 = jnp.float32
    row = lambda v: v.reshape(1, _D).astype(f32)

    w0 = p_ge["Ws"][0]
    w0i, w0j, w0e = w0[:_D], w0[_D:2 * _D], w0[2 * _D:]
    w0e_eff = p_e["ln_g"][:, None] * w0e
    c0 = (p_e["ln_b"] @ w0e + p_ge["bs"][0]).reshape(1, _D)

    mesh8 = jnp.pad(input_mesh_nodes, ((0, 0), (0, 5)))
    wm0 = jnp.pad(p_m["Ws"][0], ((0, 5), (0, 0)))

    mesh_emb, table_a = _run_rows(
        _mesh_body, _NM, _BLN, 1, (_D, _W),
        (mesh8, wm0, row(p_m["bs"][0]), p_m["Ws"][1], row(p_m["bs"][1]),
         row(p_m["ln_g"]), row(p_m["ln_b"]), w0i, c0))

    (table_b,) = _run_rows(
        _gridhead_body, _NM, _BLN, 1, (_W,),
        (input_grid_nodes[:_NM], p_g["Ws"][0], row(p_g["bs"][0]),
         p_g["Ws"][1], row(p_g["bs"][1]), row(p_g["ln_g"]),
         row(p_g["ln_b"]), w0j))

    shp = (_NW, _CHUNKS, _CH)
    send = jnp.pad(edge_index[0], (0, _NEP - _NE)).reshape(shp)
    recv = jnp.pad(edge_index[1], (0, _NEP - _NE)).reshape(shp)
    recv_s = jnp.pad(edge_index[1], (0, _NEP - _NE),
                     constant_values=_NM).reshape(shp)
    g_edges = _sc_gather(table_a, table_b, recv, send)

    eargs = (input_edge_attr, g_edges, p_e["Ws"][0], row(p_e["bs"][0]),
             p_e["Ws"][1], row(p_e["bs"][1]), w0e_eff, p_ge["Ws"][1],
             row(p_ge["bs"][1]), row(p_ge["ln_g"]), row(p_ge["ln_b"]))
    in_specs = [pl.BlockSpec((_BLE, 4), lambda i: (i, 0)),
                pl.BlockSpec((_BLE, _W), lambda i: (i, 0))]
    in_specs += [_wspec(a.shape) for a in eargs[2:]]
    msg = pl.pallas_call(
        _edge_msg_body,
        grid=(_NE // _BLE,),
        in_specs=in_specs,
        out_shape=jax.ShapeDtypeStruct((_NEP, _D), jnp.float32),
        out_specs=pl.BlockSpec((_BLE, _D), lambda i: (i, 0)),
    )(*eargs)

    partials = _sc_scatter(msg, recv_s, jnp.zeros((_APT, _D), f32))

    wn0 = p_gn["Ws"][0]
    (latent_mesh,) = _run_rows(
        _mesh_out_body, _NM, _BLN, 3, (_D,),
        (mesh_emb, partials[0, :_NM, :_D], partials[1, :_NM, :_D],
         wn0[:_D], wn0[_D:], row(p_gn["bs"][0]), p_gn["Ws"][1],
         row(p_gn["bs"][1]), row(p_gn["ln_g"]), row(p_gn["ln_b"])))

    (latent_grid,) = _run_rows(
        _grid_body, _NG, _BLN, 1, (_D,),
        (input_grid_nodes, p_g["Ws"][0], row(p_g["bs"][0]), p_g["Ws"][1],
         row(p_g["bs"][1]), row(p_g["ln_g"]), row(p_g["ln_b"]),
         p_gf["Ws"][0], row(p_gf["bs"][0]), p_gf["Ws"][1], row(p_gf["bs"][1]),
         row(p_gf["ln_g"]), row(p_gf["ln_b"])))

    return (latent_grid, latent_mesh)


# speedup vs baseline: 1.0853x; 1.0477x over previous
import functools

import jax
import jax.numpy as jnp
from jax import lax
from jax.experimental import pallas as pl
from jax.experimental.pallas import tpu as pltpu
from jax.experimental.pallas import tpu_sc as plsc

_NG = 100000
_NM = 10000
_NE = 500000
_D = 64
_W = 128

_NC = 2
_NS = 16
_NW = _NC * _NS
_CH = 128
_NEH = 250000
_PER_W = 7936
_CHUNKS = _PER_W // _CH
_NEP = _PER_W * _NW
_ACC = 10240
_APT = _ACC // _NS

_BLE = 2000
_BLN = 2000


def _ln(h):
    mu = jnp.mean(h, axis=-1, keepdims=True)
    var = jnp.mean((h - mu) ** 2, axis=-1, keepdims=True)
    return (h - mu) * lax.rsqrt(var + 1e-5)


def _dot(a, b):
    return jnp.dot(a, b, preferred_element_type=jnp.float32)


def _wspec(shape):
    return pl.BlockSpec(shape, lambda i: (0,) * len(shape))


def _mesh_body(x_ref, w0, b0, w1, b1, g, b, w0i, c0, me_ref, ta_ref):
    x = x_ref[...]
    h = jnp.maximum(_dot(x, w0[...]) + b0[...], 0.0)
    h = _dot(h, w1[...]) + b1[...]
    me = _ln(h) * g[...] + b[...]
    me_ref[...] = me
    a = _dot(me, w0i[...]) + c0[...]
    ta_ref[...] = jnp.concatenate([a, jnp.zeros_like(a)], axis=1)


def _gridhead_body(x_ref, w0, b0, w1, b1, g, b, w0j, tb_ref):
    x = x_ref[...]
    h = jnp.maximum(_dot(x, w0[...]) + b0[...], 0.0)
    h = _dot(h, w1[...]) + b1[...]
    ge = _ln(h) * g[...] + b[...]
    tb = _dot(ge, w0j[...])
    tb_ref[...] = jnp.concatenate([jnp.zeros_like(tb), tb], axis=1)


def _edge_msg_body(e_ref, g_in_ref, we0, be0, we1, be1, w0e, u1, u1b, gg, gb,
                   out_ref):
    e = e_ref[...]
    h = jnp.maximum(_dot(e, we0[...]) + be0[...], 0.0)
    h = _dot(h, we1[...]) + be1[...]
    nh = _ln(h)
    gv = g_in_ref[...]
    h1 = _dot(nh, w0e[...]) + gv[:, :_D] + gv[:, _D:]
    y = jnp.maximum(h1, 0.0)
    y = _dot(y, u1[...]) + u1b[...]
    msg = _ln(y) * gg[...] + gb[...]
    out_ref[...] = jnp.concatenate([msg, jnp.zeros_like(msg)], axis=1)


def _grid_body(x_ref, w0, b0, w1, b1, g, b, v0, d0, v1, d1, gf, bf, out_ref):
    x = x_ref[...]
    h = jnp.maximum(_dot(x, w0[...]) + b0[...], 0.0)
    h = _dot(h, w1[...]) + b1[...]
    ge = _ln(h) * g[...] + b[...]
    f = jnp.maximum(_dot(ge, v0[...]) + d0[...], 0.0)
    f = _dot(f, v1[...]) + d1[...]
    out_ref[...] = ge + _ln(f) * gf[...] + bf[...]


def _mesh_out_body(me_ref, p0_ref, p1_ref, p2_ref, p3_ref,
                   w0m, w0a, b0, w1, b1, g, b, out_ref):
    me = me_ref[...]
    aggr = (p0_ref[...] + p1_ref[...]) + (p2_ref[...] + p3_ref[...])
    h = jnp.maximum(_dot(me, w0m[...]) + _dot(aggr, w0a[...]) + b0[...], 0.0)
    h = _dot(h, w1[...]) + b1[...]
    out_ref[...] = me + _ln(h) * g[...] + b[...]


def _run_rows(body, n_rows, blk, n_blocked, out_widths, args):
    in_specs = []
    for i, a in enumerate(args):
        if i < n_blocked:
            in_specs.append(pl.BlockSpec((blk, a.shape[1]), lambda i: (i, 0)))
        else:
            in_specs.append(_wspec(a.shape))
    out = pl.pallas_call(
        body,
        grid=(n_rows // blk,),
        in_specs=in_specs,
        out_shape=[jax.ShapeDtypeStruct((n_rows, w), jnp.float32)
                   for w in out_widths],
        out_specs=[pl.BlockSpec((blk, w), lambda i: (i, 0))
                   for w in out_widths],
    )(*args)
    return out


_SC_MESH = plsc.VectorSubcoreMesh(core_axis_name="c", subcore_axis_name="s")


@functools.partial(
    pl.kernel,
    out_type=jax.ShapeDtypeStruct((_NEP, _W), jnp.float32),
    mesh=_SC_MESH,
    scratch_types=[
        pltpu.VMEM((_CHUNKS, _CH), jnp.int32),
        pltpu.VMEM((_CHUNKS, _CH), jnp.int32),
        pltpu.VMEM((_CH, _W), jnp.float32),
        pltpu.VMEM((_CH, _W), jnp.float32),
        pltpu.VMEM((_CH, _W), jnp.float32),
        pltpu.VMEM((_CH, _W), jnp.float32),
        pltpu.SemaphoreType.DMA,
        pltpu.SemaphoreType.DMA,
    ],
)
def _sc_gather(ta_hbm, tb_hbm, recv_hbm, send_hbm, g_hbm,
               ridx_all, sidx_all, ra0, ra1, rb0, rb1, sem_a, sem_b):
    c = lax.axis_index("c")
    s = lax.axis_index("s")
    w = s * _NC + c
    base = w * _PER_W
    ta = ta_hbm
    tb = tb_hbm
    pltpu.sync_copy(recv_hbm.at[w], ridx_all)
    pltpu.sync_copy(send_hbm.at[w], sidx_all)
    bufs_a = (ra0, ra1)
    bufs_b = (rb0, rb1)
    pltpu.async_copy(ta.at[ridx_all.at[0]], ra0, sem_a)
    pltpu.async_copy(tb.at[sidx_all.at[0]], rb0, sem_b)

    def body(i, carry):
        for b in range(2):
            j = i * 2 + b
            ra, rb = bufs_a[b], bufs_b[b]
            na, nb = bufs_a[1 - b], bufs_b[1 - b]
            jn = j + 1

            @pl.when(jn < _CHUNKS)
            def _():
                pltpu.async_copy(ta.at[ridx_all.at[jn]], na, sem_a)
                pltpu.async_copy(tb.at[sidx_all.at[jn]], nb, sem_b)

            pltpu.make_async_copy(ta.at[ridx_all.at[j]], ra, sem_a).wait()
            pltpu.make_async_copy(tb.at[sidx_all.at[j]], rb, sem_b).wait()

            def merge(r, cc):
                for k in range(_D // 16):
                    sl = pl.ds(_D + 16 * k, 16)
                    ra[r, sl] = rb[r, sl]
                return cc

            lax.fori_loop(0, _CH, merge, 0)
            pltpu.sync_copy(ra, g_hbm.at[pl.ds(base + j * _CH, _CH)])
        return carry

    lax.fori_loop(0, _CHUNKS // 2, body, 0)


@functools.partial(
    pl.kernel,
    out_type=jax.ShapeDtypeStruct((_NC, _ACC, _W), jnp.float32),
    mesh=_SC_MESH,
    scratch_types=[
        pltpu.VMEM((_CHUNKS, _CH), jnp.int32),
        pltpu.VMEM((_CH, _W), jnp.float32),
        pltpu.VMEM((_CH, _W), jnp.float32),
        pltpu.VMEM_SHARED((_ACC, _W), jnp.float32),
        pltpu.SemaphoreType.DMA,
    ],
)
def _sc_scatter(msg_hbm, recv_hbm, zeros_hbm, out_hbm,
                ridx_all, r0, r1, acc, sem_m):
    c = lax.axis_index("c")
    s = lax.axis_index("s")
    pltpu.sync_copy(zeros_hbm, acc.at[pl.ds(s * _APT, _APT)])
    w = s * _NC + c
    base = w * _PER_W
    pltpu.sync_copy(recv_hbm.at[w], ridx_all)
    plsc.subcore_barrier()
    bufs = (r0, r1)
    pltpu.async_copy(msg_hbm.at[pl.ds(base, _CH)], r0, sem_m)

    def body(i, carry):
        for b in range(2):
            j = i * 2 + b
            rcur = bufs[b]
            rnext = bufs[1 - b]
            jn = j + 1

            @pl.when(jn < _CHUNKS)
            def _():
                pltpu.async_copy(msg_hbm.at[pl.ds(base + jn * _CH, _CH)],
                                 rnext, sem_m)

            pltpu.make_async_copy(msg_hbm.at[pl.ds(base, _CH)], rcur,
                                  sem_m).wait()
            pltpu.sync_copy(rcur, acc.at[ridx_all.at[j]], add=True)
        return carry

    lax.fori_loop(0, _CHUNKS // 2, body, 0)
    plsc.subcore_barrier()
    pltpu.sync_copy(acc.at[pl.ds(s * _APT, _APT)],
                    out_hbm.at[c].at[pl.ds(s * _APT, _APT)])


def kernel(input_grid_nodes, input_mesh_nodes, input_edge_attr, params,
           edge_index):
    p_g = params["grid_mlp"]
    p_m = params["mesh_mlp"]
    p_e = params["edges_mlp"]
    p_ge = params["gnn_edges"]
    p_gn = params["gnn_nodes"]
    p_gf = params["grid_mlp_final"]

    f32 = jnp.float32
    row = lambda v: v.reshape(1, _D).astype(f32)

    w0 = p_ge["Ws"][0]
    w0i, w0j, w0e = w0[:_D], w0[_D:2 * _D], w0[2 * _D:]
    w0e_eff = p_e["ln_g"][:, None] * w0e
    c0 = (p_e["ln_b"] @ w0e + p_ge["bs"][0]).reshape(1, _D)

    mesh8 = jnp.pad(input_mesh_nodes, ((0, 0), (0, 5)))
    wm0 = jnp.pad(p_m["Ws"][0], ((0, 5), (0, 0)))

    mesh_emb, table_a = _run_rows(
        _mesh_body, _NM, _BLN, 1, (_D, _W),
        (mesh8, wm0, row(p_m["bs"][0]), p_m["Ws"][1], row(p_m["bs"][1]),
         row(p_m["ln_g"]), row(p_m["ln_b"]), w0i, c0))

    (table_b,) = _run_rows(
        _gridhead_body, _NM, _BLN, 1, (_W,),
        (input_grid_nodes[:_NM], p_g["Ws"][0], row(p_g["bs"][0]),
         p_g["Ws"][1], row(p_g["bs"][1]), row(p_g["ln_g"]),
         row(p_g["ln_b"]), w0j))

    shp = (_NW, _CHUNKS, _CH)
    n_blk = _NEH // _BLE
    partials = []
    for h in range(2):
        lo, hi = h * _NEH, (h + 1) * _NEH
        pad = (0, _NEP - _NEH)
        send_h = jnp.pad(edge_index[0][lo:hi], pad).reshape(shp)
        recv_h = jnp.pad(edge_index[1][lo:hi], pad).reshape(shp)
        recv_sh = jnp.pad(edge_index[1][lo:hi], pad,
                          constant_values=_NM).reshape(shp)
        g_edges = _sc_gather(table_a, table_b, recv_h, send_h)

        eargs = (input_edge_attr, g_edges, p_e["Ws"][0], row(p_e["bs"][0]),
                 p_e["Ws"][1], row(p_e["bs"][1]), w0e_eff, p_ge["Ws"][1],
                 row(p_ge["bs"][1]), row(p_ge["ln_g"]), row(p_ge["ln_b"]))
        in_specs = [
            pl.BlockSpec((_BLE, 4), lambda i, o=h * n_blk: (i + o, 0)),
            pl.BlockSpec((_BLE, _W), lambda i: (i, 0))]
        in_specs += [_wspec(a.shape) for a in eargs[2:]]
        msg = pl.pallas_call(
            _edge_msg_body,
            grid=(n_blk,),
            in_specs=in_specs,
            out_shape=jax.ShapeDtypeStruct((_NEP, _W), jnp.float32),
            out_specs=pl.BlockSpec((_BLE, _W), lambda i: (i, 0)),
        )(*eargs)

        partials.append(_sc_scatter(msg, recv_sh, jnp.zeros((_APT, _W), f32)))

    wn0 = p_gn["Ws"][0]
    (latent_mesh,) = _run_rows(
        _mesh_out_body, _NM, _BLN, 5, (_D,),
        (mesh_emb, partials[0][0, :_NM, :_D], partials[0][1, :_NM, :_D],
         partials[1][0, :_NM, :_D], partials[1][1, :_NM, :_D],
         wn0[:_D], wn0[_D:], row(p_gn["bs"][0]), p_gn["Ws"][1],
         row(p_gn["bs"][1]), row(p_gn["ln_g"]), row(p_gn["ln_b"])))

    (latent_grid,) = _run_rows(
        _grid_body, _NG, _BLN, 1, (_D,),
        (input_grid_nodes, p_g["Ws"][0], row(p_g["bs"][0]), p_g["Ws"][1],
         row(p_g["bs"][1]), row(p_g["ln_g"]), row(p_g["ln_b"]),
         p_gf["Ws"][0], row(p_gf["bs"][0]), p_gf["Ws"][1], row(p_gf["bs"][1]),
         row(p_gf["ln_g"]), row(p_gf["ln_b"])))

    return (latent_grid, latent_mesh)
